# baseline (speedup 1.0000x reference)
import functools

import jax
import jax.numpy as jnp
from jax import lax
from jax.experimental import pallas as pl
from jax.experimental.pallas import tpu as pltpu
from jax.experimental.pallas import tpu_sc as plsc

_ETA_MAX = 0.5


def _make_sc1(B, twoD, NC, NS):
    NW = NC * NS
    BPW = B // NW
    CH = 128
    mesh = plsc.VectorSubcoreMesh(core_axis_name="c", subcore_axis_name="s")

    @functools.partial(
        pl.kernel,
        mesh=mesh,
        out_type=[
            jax.ShapeDtypeStruct((B, 128), jnp.int32),
            jax.ShapeDtypeStruct((B, twoD), jnp.float32),
        ],
        scratch_types=[
            pltpu.VMEM((BPW,), jnp.int32),
            pltpu.VMEM((BPW, 128), jnp.int32),
            pltpu.VMEM((BPW, twoD), jnp.float32),
            pltpu.SemaphoreType.DMA,
        ],
    )
    def sc1(anchor_h, combo_h, emb_h,
            aall_o, ei_o,
            aid_v, all_v, ei_v, sem):
        wid = lax.axis_index("s") * NC + lax.axis_index("c")
        base = wid * BPW
        pltpu.sync_copy(anchor_h.at[pl.ds(base, BPW)], aid_v)
        cps = []
        for j in range(BPW // CH):
            idx = aid_v.at[pl.ds(j * CH, CH)]
            sl = pl.ds(j * CH, CH)
            cps.append(pltpu.async_copy(combo_h.at[idx], all_v.at[sl], sem))
            cps.append(pltpu.async_copy(emb_h.at[idx], ei_v.at[sl], sem))
        for c in cps:
            c.wait()
        out = pl.ds(base, BPW)
        pltpu.sync_copy(all_v, aall_o.at[out])
        pltpu.sync_copy(ei_v, ei_o.at[out])

    return sc1


def _make_sc2(BK, twoD, D, NC, NS):
    NW = NC * NS
    RPW = BK // NW
    CH = 128
    NIT = RPW // CH
    mesh = plsc.VectorSubcoreMesh(core_axis_name="c", subcore_axis_name="s")

    @functools.partial(
        pl.kernel,
        mesh=mesh,
        out_type=[
            jax.ShapeDtypeStruct((BK, twoD), jnp.float32),
            jax.ShapeDtypeStruct((BK, D), jnp.float32),
        ],
        scratch_types=[
            pltpu.VMEM((CH,), jnp.int32),
            pltpu.VMEM((CH,), jnp.int32),
            pltpu.VMEM((CH, twoD), jnp.float32),
            pltpu.VMEM((CH, D), jnp.float32),
            pltpu.SemaphoreType.DMA,
        ],
    )
    def sc2(aent_h, arel_h, emb_h, ph_h,
            ej_o, ph_o,
            ie_v, ir_v, ej_v, ph_v, sem):
        wid = lax.axis_index("s") * NC + lax.axis_index("c")
        base = wid * RPW

        def body(i, carry):
            r0 = base + i * CH
            sl = pl.ds(r0, CH)
            pltpu.sync_copy(aent_h.at[sl], ie_v)
            pltpu.sync_copy(arel_h.at[sl], ir_v)
            c1 = pltpu.async_copy(emb_h.at[ie_v], ej_v, sem)
            c2 = pltpu.async_copy(ph_h.at[ir_v], ph_v, sem)
            c1.wait()
            c2.wait()
            pltpu.sync_copy(ej_v, ej_o.at[sl])
            pltpu.sync_copy(ph_v, ph_o.at[sl])
            return carry

        lax.fori_loop(0, NIT, body, 0)

    return sc2


def _tc_body(ej_ref, ph_ref, ei_ref, aall_ref, wq_ref, wk_ref,
             par_ref, out_ref, *, BB, K, D, A, scale):
    twoD = 2 * D
    ej = ej_ref[...]
    ph = ph_ref[...]
    cosp = jnp.cos(ph)
    sinp = jnp.sin(ph)
    dm = aall_ref[:, K:2 * K] >> 10
    dir_i = dm & 1
    msk = (dm >> 1) != 0
    re_j = ej[:, :D]
    im_j = ej[:, D:]
    re_m = re_j * cosp - im_j * sinp
    im_m = re_j * sinp + im_j * cosp
    hat = jnp.concatenate([re_m, im_m], axis=1)

    ei = ei_ref[...]
    q = lax.dot_general(ei, wq_ref[...], (((1,), (1,)), ((), ())),
                        preferred_element_type=jnp.float32)
    kk = lax.dot_general(hat, wk_ref[...], (((1,), (1,)), ((), ())),
                         preferred_element_type=jnp.float32)
    k3 = kk.reshape(BB, K, A)
    logits = jnp.sum(q[:, None, :] * k3, axis=-1) * (1.0 / scale)

    rb = lax.bitcast_convert_type(aall_ref[:, 2 * K:3 * K],
                                  jnp.float32)
    d0 = par_ref[1, 0]
    d1 = par_ref[1, 1]
    s0 = par_ref[1, 2]
    w = par_ref[1, 3]
    dirf = dir_i.astype(jnp.float32)
    logits = logits + rb + d0 + dirf * (d1 - d0)
    logits = jnp.where(msk, logits, -10000.0)
    m = jnp.max(logits, axis=1, keepdims=True)
    ex = jnp.exp(logits - m)
    attn = ex / jnp.sum(ex, axis=1, keepdims=True)

    hat3 = hat.reshape(BB, K, twoD)
    delta = jnp.sum(attn[:, :, None] * hat3, axis=1)
    av = par_ref[0:1, :D]
    a2 = jnp.concatenate([av, av], axis=1)
    deltam = delta * a2

    fbits = aall_ref[:, 3 * K:3 * K + 1]
    f = lax.bitcast_convert_type(fbits, jnp.float32)
    logf = jnp.log1p(f)
    has = jnp.any(msk, axis=1, keepdims=True)
    eta = _ETA_MAX * jax.nn.sigmoid(s0 - w * logf)
    eta = eta * has.astype(jnp.float32)
    out_ref[...] = ei + eta * (deltam - ei)


def kernel(anchor_ids, entity_embedding, relation_phase, nbr_ent, nbr_rel,
           nbr_dir, nbr_mask, freq, a_vec, eta_raw, w_raw, b, Wq, Wk,
           rel_bias, dir_bias):
    N, twoD = entity_embedding.shape
    D = twoD // 2
    B = anchor_ids.shape[0]
    K = nbr_ent.shape[1]
    A = Wq.shape[0]
    BK = B * K
    scale = max(A ** 0.5, 1e-06)

    info = plsc.get_sparse_core_info()
    NC, NS = info.num_cores, info.num_subcores

    aid = anchor_ids.astype(jnp.int32)
    nrel = nbr_rel.astype(jnp.int32)
    ndir = nbr_dir.astype(jnp.int32)
    dm = ndir + 2 * nbr_mask.astype(jnp.int32)
    R = relation_phase.shape[0]
    rel_eff = nrel + R * ndir
    rel_dm = rel_eff | (dm << 10)
    bias_bits = lax.bitcast_convert_type(
        jnp.take(rel_bias[:, 0].astype(jnp.float32), nrel, axis=0),
        jnp.int32)
    fbits = lax.bitcast_convert_type(freq.astype(jnp.float32),
                                     jnp.int32)[:, None]
    combo = jnp.concatenate(
        [nbr_ent.astype(jnp.int32), rel_dm, bias_bits,
         jnp.broadcast_to(fbits, (N, K))], axis=1)
    ph_tab = jnp.concatenate([relation_phase, -relation_phase], axis=0)

    sc1 = _make_sc1(B, twoD, NC, NS)
    a_all, e_i = sc1(aid, combo, entity_embedding)

    aent_f = a_all[:, :K].reshape(BK)
    arel_f = (a_all[:, K:2 * K] & 1023).reshape(BK)
    sc2 = _make_sc2(BK, twoD, D, NC, NS)
    e_j, ph = sc2(aent_f, arel_f, entity_embedding, ph_tab)

    w_sp = jax.nn.softplus(w_raw)
    s0 = eta_raw + b
    row1 = jnp.zeros((D,), jnp.float32)
    row1 = row1.at[0].set(dir_bias[0, 0]).at[1].set(dir_bias[1, 0])
    row1 = row1.at[2].set(s0).at[3].set(w_sp)
    par = jnp.concatenate(
        [a_vec[None, :].astype(jnp.float32), row1[None, :],
         jnp.zeros((6, D), jnp.float32)], axis=0)

    BB = 128
    grid = (B // BB,)
    body = functools.partial(_tc_body, BB=BB, K=K, D=D, A=A, scale=scale)
    out = pl.pallas_call(
        body,
        grid=grid,
        in_specs=[
            pl.BlockSpec((BB * K, twoD), lambda i: (i, 0)),
            pl.BlockSpec((BB * K, D), lambda i: (i, 0)),
            pl.BlockSpec((BB, twoD), lambda i: (i, 0)),
            pl.BlockSpec((BB, 4 * K), lambda i: (i, 0)),
            pl.BlockSpec((A, twoD), lambda i: (0, 0)),
            pl.BlockSpec((A, twoD), lambda i: (0, 0)),
            pl.BlockSpec((8, D), lambda i: (0, 0)),
        ],
        out_specs=pl.BlockSpec((BB, twoD), lambda i: (i, 0)),
        out_shape=jax.ShapeDtypeStruct((B, twoD), jnp.float32),
    )(e_j, ph, e_i, a_all, Wq, Wk, par)
    return out


# speedup vs baseline: 10.4178x; 10.4178x over previous
import functools

import jax
import jax.numpy as jnp
from jax import lax
from jax.experimental import pallas as pl
from jax.experimental.pallas import tpu as pltpu
from jax.experimental.pallas import tpu_sc as plsc

_ETA_MAX = 0.5


def _make_sc1(B, twoD, NC, NS):
    NW = NC * NS
    BPW = B // NW
    CH = 128
    mesh = plsc.VectorSubcoreMesh(core_axis_name="c", subcore_axis_name="s")

    @functools.partial(
        pl.kernel,
        mesh=mesh,
        out_type=[
            jax.ShapeDtypeStruct((B, 128), jnp.int32),
            jax.ShapeDtypeStruct((B, twoD), jnp.float32),
        ],
        scratch_types=[
            pltpu.VMEM((BPW,), jnp.int32),
            pltpu.VMEM((BPW, 128), jnp.int32),
            pltpu.VMEM((BPW, twoD), jnp.float32),
            pltpu.SemaphoreType.DMA,
        ],
    )
    def sc1(anchor_h, combo_h, emb_h,
            aall_o, ei_o,
            aid_v, all_v, ei_v, sem):
        wid = lax.axis_index("s") * NC + lax.axis_index("c")
        base = wid * BPW
        pltpu.sync_copy(anchor_h.at[pl.ds(base, BPW)], aid_v)
        cps = []
        for j in range(BPW // CH):
            idx = aid_v.at[pl.ds(j * CH, CH)]
            sl = pl.ds(j * CH, CH)
            cps.append(pltpu.async_copy(combo_h.at[idx], all_v.at[sl], sem))
            cps.append(pltpu.async_copy(emb_h.at[idx], ei_v.at[sl], sem))
        for c in cps:
            c.wait()
        out = pl.ds(base, BPW)
        pltpu.sync_copy(all_v, aall_o.at[out])
        pltpu.sync_copy(ei_v, ei_o.at[out])

    return sc1


def _make_sc2(BK, twoD, D, NC, NS):
    NW = NC * NS
    RPW = BK // NW
    CH = 128
    NIT = RPW // CH
    mesh = plsc.VectorSubcoreMesh(core_axis_name="c", subcore_axis_name="s")

    @functools.partial(
        pl.kernel,
        mesh=mesh,
        out_type=[
            jax.ShapeDtypeStruct((BK, twoD), jnp.float32),
            jax.ShapeDtypeStruct((BK, twoD), jnp.float32),
        ],
        scratch_types=[
            pltpu.VMEM((CH,), jnp.int32),
            pltpu.VMEM((CH,), jnp.int32),
            pltpu.VMEM((CH, twoD), jnp.float32),
            pltpu.VMEM((CH, twoD), jnp.float32),
            pltpu.SemaphoreType.DMA,
        ],
    )
    def sc2(aent_h, arel_h, emb_h, ph_h,
            ej_o, ph_o,
            ie_v, ir_v, ej_v, ph_v, sem):
        wid = lax.axis_index("s") * NC + lax.axis_index("c")
        base = wid * RPW

        def body(i, carry):
            r0 = base + i * CH
            sl = pl.ds(r0, CH)
            pltpu.sync_copy(aent_h.at[sl], ie_v)
            pltpu.sync_copy(arel_h.at[sl], ir_v)
            c1 = pltpu.async_copy(emb_h.at[ie_v], ej_v, sem)
            c2 = pltpu.async_copy(ph_h.at[ir_v], ph_v, sem)
            c1.wait()
            c2.wait()
            pltpu.sync_copy(ej_v, ej_o.at[sl])
            pltpu.sync_copy(ph_v, ph_o.at[sl])
            return carry

        lax.fori_loop(0, NIT, body, 0)

    return sc2


def _tc_body(ej_ref, ph_ref, ei_ref, aall_ref, wq_ref, wk_ref,
             par_ref, out_ref, *, BB, K, D, A, scale):
    twoD = 2 * D
    ej = ej_ref[...]
    ph = ph_ref[:, :D]
    cosp = jnp.cos(ph)
    sinp = jnp.sin(ph)
    dm = aall_ref[:, K:2 * K] >> 10
    dir_i = dm & 1
    msk = (dm >> 1) != 0
    re_j = ej[:, :D]
    im_j = ej[:, D:]
    re_m = re_j * cosp - im_j * sinp
    im_m = re_j * sinp + im_j * cosp
    hat = jnp.concatenate([re_m, im_m], axis=1)

    ei = ei_ref[...]
    q = lax.dot_general(ei, wq_ref[...], (((1,), (1,)), ((), ())),
                        preferred_element_type=jnp.float32)
    kk = lax.dot_general(hat, wk_ref[...], (((1,), (1,)), ((), ())),
                         preferred_element_type=jnp.float32)
    k3 = kk.reshape(BB, K, A)
    logits = jnp.sum(q[:, None, :] * k3, axis=-1) * (1.0 / scale)

    rb = jnp.sum(ph_ref[:, D:].reshape(BB, K, D), axis=-1) * (1.0 / D)
    d0 = par_ref[1, 0]
    d1 = par_ref[1, 1]
    s0 = par_ref[1, 2]
    w = par_ref[1, 3]
    dirf = dir_i.astype(jnp.float32)
    logits = logits + rb + d0 + dirf * (d1 - d0)
    logits = jnp.where(msk, logits, -10000.0)
    m = jnp.max(logits, axis=1, keepdims=True)
    ex = jnp.exp(logits - m)
    attn = ex / jnp.sum(ex, axis=1, keepdims=True)

    hat3 = hat.reshape(BB, K, twoD)
    delta = jnp.sum(attn[:, :, None] * hat3, axis=1)
    av = par_ref[0:1, :D]
    a2 = jnp.concatenate([av, av], axis=1)
    deltam = delta * a2

    fbits = aall_ref[:, 3 * K:3 * K + 1]
    f = lax.bitcast_convert_type(fbits, jnp.float32)
    logf = jnp.log1p(f)
    has = jnp.any(msk, axis=1, keepdims=True)
    eta = _ETA_MAX * jax.nn.sigmoid(s0 - w * logf)
    eta = eta * has.astype(jnp.float32)
    out_ref[...] = ei + eta * (deltam - ei)


def kernel(anchor_ids, entity_embedding, relation_phase, nbr_ent, nbr_rel,
           nbr_dir, nbr_mask, freq, a_vec, eta_raw, w_raw, b, Wq, Wk,
           rel_bias, dir_bias):
    N, twoD = entity_embedding.shape
    D = twoD // 2
    B = anchor_ids.shape[0]
    K = nbr_ent.shape[1]
    A = Wq.shape[0]
    BK = B * K
    scale = max(A ** 0.5, 1e-06)

    info = plsc.get_sparse_core_info()
    NC, NS = info.num_cores, info.num_subcores

    aid = anchor_ids.astype(jnp.int32)
    nrel = nbr_rel.astype(jnp.int32)
    ndir = nbr_dir.astype(jnp.int32)
    dm = ndir + 2 * nbr_mask.astype(jnp.int32)
    R = relation_phase.shape[0]
    rel_eff = nrel + R * ndir
    rel_dm = rel_eff | (dm << 10)
    fbits = lax.bitcast_convert_type(freq.astype(jnp.float32),
                                     jnp.int32)[:, None]
    fcols = jnp.broadcast_to(fbits, (N, K))
    combo = jnp.concatenate(
        [nbr_ent.astype(jnp.int32), rel_dm, fcols, fcols],
        axis=1)
    bias128 = jnp.broadcast_to(rel_bias.astype(jnp.float32), (R, D))
    ph_tab = jnp.concatenate(
        [jnp.concatenate([relation_phase, bias128], axis=1),
         jnp.concatenate([-relation_phase, bias128], axis=1)],
        axis=0)

    sc1 = _make_sc1(B, twoD, NC, NS)
    a_all, e_i = sc1(aid, combo, entity_embedding)

    aent_f = a_all[:, :K].reshape(BK)
    arel_f = (a_all[:, K:2 * K] & 1023).reshape(BK)
    sc2 = _make_sc2(BK, twoD, D, NC, NS)
    e_j, ph = sc2(aent_f, arel_f, entity_embedding, ph_tab)

    w_sp = jax.nn.softplus(w_raw)
    s0 = eta_raw + b
    row1 = jnp.zeros((D,), jnp.float32)
    row1 = row1.at[0].set(dir_bias[0, 0]).at[1].set(dir_bias[1, 0])
    row1 = row1.at[2].set(s0).at[3].set(w_sp)
    par = jnp.concatenate(
        [a_vec[None, :].astype(jnp.float32), row1[None, :],
         jnp.zeros((6, D), jnp.float32)], axis=0)

    BB = 128
    grid = (B // BB,)
    body = functools.partial(_tc_body, BB=BB, K=K, D=D, A=A, scale=scale)
    out = pl.pallas_call(
        body,
        grid=grid,
        in_specs=[
            pl.BlockSpec((BB * K, twoD), lambda i: (i, 0)),
            pl.BlockSpec((BB * K, twoD), lambda i: (i, 0)),
            pl.BlockSpec((BB, twoD), lambda i: (i, 0)),
            pl.BlockSpec((BB, 4 * K), lambda i: (i, 0)),
            pl.BlockSpec((A, twoD), lambda i: (0, 0)),
            pl.BlockSpec((A, twoD), lambda i: (0, 0)),
            pl.BlockSpec((8, D), lambda i: (0, 0)),
        ],
        out_specs=pl.BlockSpec((BB, twoD), lambda i: (i, 0)),
        out_shape=jax.ShapeDtypeStruct((B, twoD), jnp.float32),
    )(e_j, ph, e_i, a_all, Wq, Wk, par)
    return out


# speedup vs baseline: 13.9038x; 1.3346x over previous
import functools

import jax
import jax.numpy as jnp
from jax import lax
from jax.experimental import pallas as pl
from jax.experimental.pallas import tpu as pltpu
from jax.experimental.pallas import tpu_sc as plsc

_ETA_MAX = 0.5


def _make_sc1(B, twoD, NC, NS):
    NW = NC * NS
    BPW = B // NW
    CH = 128
    mesh = plsc.VectorSubcoreMesh(core_axis_name="c", subcore_axis_name="s")

    @functools.partial(
        pl.kernel,
        mesh=mesh,
        out_type=[
            jax.ShapeDtypeStruct((B, 128), jnp.int32),
            jax.ShapeDtypeStruct((B, twoD), jnp.float32),
        ],
        scratch_types=[
            pltpu.VMEM((BPW,), jnp.int32),
            pltpu.VMEM((BPW, 128), jnp.int32),
            pltpu.VMEM((BPW, twoD), jnp.float32),
            pltpu.SemaphoreType.DMA,
        ],
    )
    def sc1(anchor_h, combo_h, emb_h,
            aall_o, ei_o,
            aid_v, all_v, ei_v, sem):
        wid = lax.axis_index("s") * NC + lax.axis_index("c")
        base = wid * BPW
        pltpu.sync_copy(anchor_h.at[pl.ds(base, BPW)], aid_v)
        cps = []
        for j in range(BPW // CH):
            idx = aid_v.at[pl.ds(j * CH, CH)]
            sl = pl.ds(j * CH, CH)
            cps.append(pltpu.async_copy(combo_h.at[idx], all_v.at[sl], sem))
            cps.append(pltpu.async_copy(emb_h.at[idx], ei_v.at[sl], sem))
        for c in cps:
            c.wait()
        out = pl.ds(base, BPW)
        pltpu.sync_copy(all_v, aall_o.at[out])
        pltpu.sync_copy(ei_v, ei_o.at[out])

    return sc1


def _make_sc2(BK, twoD, D, NC, NS):
    NW = NC * NS
    RPW = BK // NW
    CH = 128
    NIT = RPW // CH
    mesh = plsc.VectorSubcoreMesh(core_axis_name="c", subcore_axis_name="s")

    @functools.partial(
        pl.kernel,
        mesh=mesh,
        out_type=[
            jax.ShapeDtypeStruct((BK, twoD), jnp.float32),
            jax.ShapeDtypeStruct((BK, 3 * D), jnp.float32),
        ],
        scratch_types=[
            pltpu.VMEM((CH,), jnp.int32),
            pltpu.VMEM((CH,), jnp.int32),
            pltpu.VMEM((CH, twoD), jnp.float32),
            pltpu.VMEM((CH, 3 * D), jnp.float32),
            pltpu.SemaphoreType.DMA,
        ],
    )
    def sc2(aent_h, arel_h, emb_h, ph_h,
            ej_o, ph_o,
            ie_v, ir_v, ej_v, ph_v, sem):
        wid = lax.axis_index("s") * NC + lax.axis_index("c")
        base = wid * RPW

        def body(i, carry):
            r0 = base + i * CH
            sl = pl.ds(r0, CH)
            pltpu.sync_copy(aent_h.at[sl], ie_v)
            pltpu.sync_copy(arel_h.at[sl], ir_v)
            c1 = pltpu.async_copy(emb_h.at[ie_v], ej_v, sem)
            c2 = pltpu.async_copy(ph_h.at[ir_v], ph_v, sem)
            c1.wait()
            c2.wait()
            pltpu.sync_copy(ej_v, ej_o.at[sl])
            pltpu.sync_copy(ph_v, ph_o.at[sl])
            return carry

        lax.fori_loop(0, NIT, body, 0)

    return sc2


def _tab_body(ph_ref, b_ref, out_ref, *, D):
    ph = ph_ref[...]
    out_ref[:, :D] = jnp.cos(ph)
    out_ref[:, D:2 * D] = jnp.sin(ph)
    out_ref[:, 2 * D:] = b_ref[...]


def _tc_body(ej_ref, ph_ref, ei_ref, aall_ref, wq_ref, wk_ref,
             par_ref, out_ref, *, BB, K, D, A, scale):
    twoD = 2 * D
    ej = ej_ref[...]
    cosp = ph_ref[:, :D]
    sinp = ph_ref[:, D:2 * D]
    dm = aall_ref[:, K:2 * K] >> 10
    dir_i = dm & 1
    msk = (dm >> 1) != 0
    re_j = ej[:, :D]
    im_j = ej[:, D:]
    re_m = re_j * cosp - im_j * sinp
    im_m = re_j * sinp + im_j * cosp
    hat = jnp.concatenate([re_m, im_m], axis=1)

    ei = ei_ref[...]
    q = lax.dot_general(ei, wq_ref[...], (((1,), (1,)), ((), ())),
                        preferred_element_type=jnp.float32)
    kk = lax.dot_general(hat, wk_ref[...], (((1,), (1,)), ((), ())),
                         preferred_element_type=jnp.float32)
    k3 = kk.reshape(BB, K, A)
    logits = jnp.sum(q[:, None, :] * k3, axis=-1) * (1.0 / scale)

    rb = jnp.sum(ph_ref[:, 2 * D:].reshape(BB, K, D), axis=-1) * (1.0 / D)
    d0 = par_ref[1, 0]
    d1 = par_ref[1, 1]
    s0 = par_ref[1, 2]
    w = par_ref[1, 3]
    dirf = dir_i.astype(jnp.float32)
    logits = logits + rb + d0 + dirf * (d1 - d0)
    logits = jnp.where(msk, logits, -10000.0)
    m = jnp.max(logits, axis=1, keepdims=True)
    ex = jnp.exp(logits - m)
    attn = ex / jnp.sum(ex, axis=1, keepdims=True)

    hat3 = hat.reshape(BB, K, twoD)
    delta = jnp.sum(attn[:, :, None] * hat3, axis=1)
    av = par_ref[0:1, :D]
    a2 = jnp.concatenate([av, av], axis=1)
    deltam = delta * a2

    fbits = aall_ref[:, 3 * K:3 * K + 1]
    f = lax.bitcast_convert_type(fbits, jnp.float32)
    logf = jnp.log1p(f)
    has = jnp.any(msk, axis=1, keepdims=True)
    eta = _ETA_MAX * jax.nn.sigmoid(s0 - w * logf)
    eta = eta * has.astype(jnp.float32)
    out_ref[...] = ei + eta * (deltam - ei)


def kernel(anchor_ids, entity_embedding, relation_phase, nbr_ent, nbr_rel,
           nbr_dir, nbr_mask, freq, a_vec, eta_raw, w_raw, b, Wq, Wk,
           rel_bias, dir_bias):
    N, twoD = entity_embedding.shape
    D = twoD // 2
    B = anchor_ids.shape[0]
    K = nbr_ent.shape[1]
    A = Wq.shape[0]
    BK = B * K
    scale = max(A ** 0.5, 1e-06)

    info = plsc.get_sparse_core_info()
    NC, NS = info.num_cores, info.num_subcores

    aid = anchor_ids.astype(jnp.int32)
    nrel = nbr_rel.astype(jnp.int32)
    ndir = nbr_dir.astype(jnp.int32)
    dm = ndir + 2 * nbr_mask.astype(jnp.int32)
    R = relation_phase.shape[0]
    rel_eff = nrel + R * ndir
    rel_dm = rel_eff | (dm << 10)
    fbits = lax.bitcast_convert_type(freq.astype(jnp.float32),
                                     jnp.int32)[:, None]
    fcols = jnp.broadcast_to(fbits, (N, K))
    combo = jnp.concatenate(
        [nbr_ent.astype(jnp.int32), rel_dm, fcols, fcols],
        axis=1)
    RP = 8 * ((2 * R + 7) // 8)
    ph_signed = jnp.concatenate([relation_phase, -relation_phase], axis=0)
    ph_signed = jnp.pad(ph_signed, ((0, RP - 2 * R), (0, 0)))
    bias128 = jnp.broadcast_to(rel_bias.astype(jnp.float32), (R, D))
    bias2 = jnp.pad(jnp.concatenate([bias128, bias128], axis=0),
                    ((0, RP - 2 * R), (0, 0)))
    ph_tab = pl.pallas_call(
        functools.partial(_tab_body, D=D),
        out_shape=jax.ShapeDtypeStruct((RP, 3 * D), jnp.float32),
    )(ph_signed, bias2)

    sc1 = _make_sc1(B, twoD, NC, NS)
    a_all, e_i = sc1(aid, combo, entity_embedding)

    aent_f = a_all[:, :K].reshape(BK)
    arel_f = (a_all[:, K:2 * K] & 1023).reshape(BK)
    sc2 = _make_sc2(BK, twoD, D, NC, NS)
    e_j, ph = sc2(aent_f, arel_f, entity_embedding, ph_tab)

    w_sp = jax.nn.softplus(w_raw)
    s0 = eta_raw + b
    row1 = jnp.zeros((D,), jnp.float32)
    row1 = row1.at[0].set(dir_bias[0, 0]).at[1].set(dir_bias[1, 0])
    row1 = row1.at[2].set(s0).at[3].set(w_sp)
    par = jnp.concatenate(
        [a_vec[None, :].astype(jnp.float32), row1[None, :],
         jnp.zeros((6, D), jnp.float32)], axis=0)

    BB = 128
    grid = (B // BB,)
    body = functools.partial(_tc_body, BB=BB, K=K, D=D, A=A, scale=scale)
    out = pl.pallas_call(
        body,
        grid=grid,
        in_specs=[
            pl.BlockSpec((BB * K, twoD), lambda i: (i, 0)),
            pl.BlockSpec((BB * K, 3 * D), lambda i: (i, 0)),
            pl.BlockSpec((BB, twoD), lambda i: (i, 0)),
            pl.BlockSpec((BB, 4 * K), lambda i: (i, 0)),
            pl.BlockSpec((A, twoD), lambda i: (0, 0)),
            pl.BlockSpec((A, twoD), lambda i: (0, 0)),
            pl.BlockSpec((8, D), lambda i: (0, 0)),
        ],
        out_specs=pl.BlockSpec((BB, twoD), lambda i: (i, 0)),
        out_shape=jax.ShapeDtypeStruct((B, twoD), jnp.float32),
    )(e_j, ph, e_i, a_all, Wq, Wk, par)
    return out


# speedup vs baseline: 14.4619x; 1.0401x over previous
import functools

import jax
import jax.numpy as jnp
from jax import lax
from jax.experimental import pallas as pl
from jax.experimental.pallas import tpu as pltpu
from jax.experimental.pallas import tpu_sc as plsc

_ETA_MAX = 0.5


def _make_sc1(B, twoD, NC, NS):
    NW = NC * NS
    BPW = B // NW
    CH = 128
    mesh = plsc.VectorSubcoreMesh(core_axis_name="c", subcore_axis_name="s")

    @functools.partial(
        pl.kernel,
        mesh=mesh,
        out_type=[
            jax.ShapeDtypeStruct((B, 128), jnp.int32),
            jax.ShapeDtypeStruct((B, twoD), jnp.float32),
        ],
        scratch_types=[
            pltpu.VMEM((BPW,), jnp.int32),
            pltpu.VMEM((BPW, 128), jnp.int32),
            pltpu.VMEM((BPW, twoD), jnp.float32),
            pltpu.SemaphoreType.DMA,
        ],
    )
    def sc1(anchor_h, combo_h, emb_h,
            aall_o, ei_o,
            aid_v, all_v, ei_v, sem):
        wid = lax.axis_index("s") * NC + lax.axis_index("c")
        base = wid * BPW
        pltpu.sync_copy(anchor_h.at[pl.ds(base, BPW)], aid_v)
        cps = []
        for j in range(BPW // CH):
            idx = aid_v.at[pl.ds(j * CH, CH)]
            sl = pl.ds(j * CH, CH)
            cps.append(pltpu.async_copy(combo_h.at[idx], all_v.at[sl], sem))
            cps.append(pltpu.async_copy(emb_h.at[idx], ei_v.at[sl], sem))
        for c in cps:
            c.wait()
        out = pl.ds(base, BPW)
        pltpu.sync_copy(all_v, aall_o.at[out])
        pltpu.sync_copy(ei_v, ei_o.at[out])

    return sc1


def _make_sc2(BK, twoD, D, NC, NS):
    NW = NC * NS
    RPW = BK // NW
    CH = 128
    NIT = RPW // CH
    mesh = plsc.VectorSubcoreMesh(core_axis_name="c", subcore_axis_name="s")

    @functools.partial(
        pl.kernel,
        mesh=mesh,
        out_type=[
            jax.ShapeDtypeStruct((BK, twoD), jnp.float32),
            jax.ShapeDtypeStruct((BK, 3 * D), jnp.float32),
        ],
        scratch_types=[
            pltpu.VMEM((CH,), jnp.int32),
            pltpu.VMEM((CH,), jnp.int32),
            pltpu.VMEM((CH, twoD), jnp.float32),
            pltpu.VMEM((CH, 3 * D), jnp.float32),
            pltpu.SemaphoreType.DMA,
        ],
    )
    def sc2(aent_h, arel_h, emb_h, ph_h,
            ej_o, ph_o,
            ie_v, ir_v, ej_v, ph_v, sem):
        wid = lax.axis_index("s") * NC + lax.axis_index("c")
        base = wid * RPW

        def body(i, carry):
            r0 = base + i * CH
            sl = pl.ds(r0, CH)
            pltpu.sync_copy(aent_h.at[sl], ie_v)
            pltpu.sync_copy(arel_h.at[sl], ir_v)
            c1 = pltpu.async_copy(emb_h.at[ie_v], ej_v, sem)
            c2 = pltpu.async_copy(ph_h.at[ir_v], ph_v, sem)
            c1.wait()
            c2.wait()
            pltpu.sync_copy(ej_v, ej_o.at[sl])
            pltpu.sync_copy(ph_v, ph_o.at[sl])
            return carry

        lax.fori_loop(0, NIT, body, 0)

    return sc2


def _tab_body(ph_ref, b_ref, out_ref, *, D):
    ph = ph_ref[...]
    out_ref[:, :D] = jnp.cos(ph)
    out_ref[:, D:2 * D] = jnp.sin(ph)
    out_ref[:, 2 * D:] = b_ref[...]


def _tc_body(ej_ref, ph_ref, ei_ref, aall_ref, wq_ref, wk_ref,
             par_ref, out_ref, *, BB, K, D, A, scale):
    twoD = 2 * D
    ej = ej_ref[...]
    cosp = ph_ref[:, :D]
    sinp = ph_ref[:, D:2 * D]
    dm = aall_ref[:, K:2 * K] >> 10
    dir_i = dm & 1
    msk = (dm >> 1) != 0
    re_j = ej[:, :D]
    im_j = ej[:, D:]
    re_m = re_j * cosp - im_j * sinp
    im_m = re_j * sinp + im_j * cosp
    hat = jnp.concatenate([re_m, im_m], axis=1)

    ei = ei_ref[...]
    q = lax.dot_general(ei, wq_ref[...], (((1,), (1,)), ((), ())),
                        preferred_element_type=jnp.float32)
    kk = lax.dot_general(hat, wk_ref[...], (((1,), (1,)), ((), ())),
                         preferred_element_type=jnp.float32)
    k3 = kk.reshape(BB, K, A)
    logits = jnp.sum(q[:, None, :] * k3, axis=-1) * (1.0 / scale)

    rb = jnp.sum(ph_ref[:, 2 * D:].reshape(BB, K, D), axis=-1) * (1.0 / D)
    d0 = par_ref[1, 0]
    d1 = par_ref[1, 1]
    s0 = par_ref[1, 2]
    w = par_ref[1, 3]
    dirf = dir_i.astype(jnp.float32)
    logits = logits + rb + d0 + dirf * (d1 - d0)
    logits = jnp.where(msk, logits, -10000.0)
    m = jnp.max(logits, axis=1, keepdims=True)
    ex = jnp.exp(logits - m)
    attn = ex / jnp.sum(ex, axis=1, keepdims=True)

    hat3 = hat.reshape(BB, K, twoD)
    delta = jnp.sum(attn[:, :, None] * hat3, axis=1)
    av = par_ref[0:1, :D]
    a2 = jnp.concatenate([av, av], axis=1)
    deltam = delta * a2

    fbits = aall_ref[:, 3 * K:3 * K + 1]
    f = lax.bitcast_convert_type(fbits, jnp.float32)
    logf = jnp.log1p(f)
    has = jnp.any(msk, axis=1, keepdims=True)
    eta = _ETA_MAX * jax.nn.sigmoid(s0 - w * logf)
    eta = eta * has.astype(jnp.float32)
    out_ref[...] = ei + eta * (deltam - ei)


def kernel(anchor_ids, entity_embedding, relation_phase, nbr_ent, nbr_rel,
           nbr_dir, nbr_mask, freq, a_vec, eta_raw, w_raw, b, Wq, Wk,
           rel_bias, dir_bias):
    N, twoD = entity_embedding.shape
    D = twoD // 2
    B = anchor_ids.shape[0]
    K = nbr_ent.shape[1]
    A = Wq.shape[0]
    BK = B * K
    scale = max(A ** 0.5, 1e-06)

    info = plsc.get_sparse_core_info()
    NC, NS = info.num_cores, info.num_subcores

    aid = anchor_ids.astype(jnp.int32)
    nrel = nbr_rel.astype(jnp.int32)
    ndir = nbr_dir.astype(jnp.int32)
    dm = ndir + 2 * nbr_mask.astype(jnp.int32)
    R = relation_phase.shape[0]
    rel_eff = nrel + R * ndir
    rel_dm = rel_eff | (dm << 10)
    fbits = lax.bitcast_convert_type(freq.astype(jnp.float32),
                                     jnp.int32)[:, None]
    fcols = jnp.broadcast_to(fbits, (N, K))
    combo = jnp.concatenate(
        [nbr_ent.astype(jnp.int32), rel_dm, fcols, fcols],
        axis=1)
    RP = 8 * ((2 * R + 7) // 8)
    ph_signed = jnp.concatenate([relation_phase, -relation_phase], axis=0)
    ph_signed = jnp.pad(ph_signed, ((0, RP - 2 * R), (0, 0)))
    bias128 = jnp.broadcast_to(rel_bias.astype(jnp.float32), (R, D))
    bias2 = jnp.pad(jnp.concatenate([bias128, bias128], axis=0),
                    ((0, RP - 2 * R), (0, 0)))
    ph_tab = pl.pallas_call(
        functools.partial(_tab_body, D=D),
        out_shape=jax.ShapeDtypeStruct((RP, 3 * D), jnp.float32),
    )(ph_signed, bias2)

    sc1 = _make_sc1(B, twoD, NC, NS)
    a_all, e_i = sc1(aid, combo, entity_embedding)

    w_sp = jax.nn.softplus(w_raw)
    s0 = eta_raw + b
    row1 = jnp.zeros((D,), jnp.float32)
    row1 = row1.at[0].set(dir_bias[0, 0]).at[1].set(dir_bias[1, 0])
    row1 = row1.at[2].set(s0).at[3].set(w_sp)
    par = jnp.concatenate(
        [a_vec[None, :].astype(jnp.float32), row1[None, :],
         jnp.zeros((6, D), jnp.float32)], axis=0)

    BB = 128
    H = 2
    B2 = B // H
    BK2 = B2 * K
    sc2 = _make_sc2(BK2, twoD, D, NC, NS)
    body = functools.partial(_tc_body, BB=BB, K=K, D=D, A=A, scale=scale)
    tc = pl.pallas_call(
        body,
        grid=(B2 // BB,),
        in_specs=[
            pl.BlockSpec((BB * K, twoD), lambda i: (i, 0)),
            pl.BlockSpec((BB * K, 3 * D), lambda i: (i, 0)),
            pl.BlockSpec((BB, twoD), lambda i: (i, 0)),
            pl.BlockSpec((BB, 4 * K), lambda i: (i, 0)),
            pl.BlockSpec((A, twoD), lambda i: (0, 0)),
            pl.BlockSpec((A, twoD), lambda i: (0, 0)),
            pl.BlockSpec((8, D), lambda i: (0, 0)),
        ],
        out_specs=pl.BlockSpec((BB, twoD), lambda i: (i, 0)),
        out_shape=jax.ShapeDtypeStruct((B2, twoD), jnp.float32),
    )

    gathered = []
    for h in range(H):
        a_h = lax.slice(a_all, (h * B2, 0), ((h + 1) * B2, a_all.shape[1]))
        aent_f = a_h[:, :K].reshape(BK2)
        arel_f = (a_h[:, K:2 * K] & 1023).reshape(BK2)
        e_j, ph = sc2(aent_f, arel_f, entity_embedding, ph_tab)
        gathered.append((a_h, e_j, ph))
    outs = []
    for h in range(H):
        a_h, e_j, ph = gathered[h]
        e_i_h = lax.slice(e_i, (h * B2, 0), ((h + 1) * B2, twoD))
        outs.append(tc(e_j, ph, e_i_h, a_h, Wq, Wk, par))
    return jnp.concatenate(outs, axis=0)
